# baseline (speedup 1.0000x reference)
import functools

import jax
import jax.numpy as jnp
from jax import lax
from jax.experimental import pallas as pl
from jax.experimental.pallas import tpu as pltpu
from jax.experimental.pallas import tpu_sc as plsc

N = 32768
F = 256
B = 16
EPS = 1e-5

NW = 32
RPW = N // NW
C = 128
NCHUNK = RPW // C
NV = F // 16
SEGPAD = 48


def _segmax_sc_body(pf_hbm, seg_hbm, out_hbm, seg_v, rows_v, acc_v, sem0, sem1):
    wid = lax.axis_index("s") * 2 + lax.axis_index("c")
    base = wid * RPW

    sems = (sem0, sem1)

    def copy(c, buf):
        return pltpu.make_async_copy(
            pf_hbm.at[pl.ds(base + c * C, C)], rows_v.at[buf], sems[buf])

    copy(0, 0).start()
    copy(1, 1).start()

    pltpu.sync_copy(seg_hbm, seg_v)

    neg = jnp.full((16,), -jnp.inf, dtype=jnp.float32)

    def init_row(b, _):
        for f in range(NV):
            acc_v[b, pl.ds(f * 16, 16)] = neg
        return 0

    lax.fori_loop(0, B, init_row, 0)

    def reduce_rows(buf, lo, hi, b):
        def rbody(r, regs):
            return tuple(
                jnp.maximum(regs[f], rows_v[buf, r, pl.ds(f * 16, 16)])
                for f in range(NV))

        init = tuple(rows_v[buf, lo, pl.ds(f * 16, 16)] for f in range(NV))
        regs = lax.fori_loop(lo + 1, hi, rbody, init)
        for f in range(NV):
            sl = pl.ds(f * 16, 16)
            acc_v[b, sl] = jnp.maximum(acc_v[b, sl], regs[f])

    def process(c, buf):
        gbase = base + c * C
        first16 = seg_v[pl.ds(0, 16)]
        next16 = seg_v[pl.ds(1, 16)]
        blo = jnp.int32(-1)
        nb = jnp.int32(0)
        for k in range(16):
            blo = blo + (first16[k] <= gbase).astype(jnp.int32)
            nb = nb + ((next16[k] > gbase) & (next16[k] < gbase + C)
                       ).astype(jnp.int32)

        @pl.when(nb == 0)
        def _fast():
            def rbody(i, regs):
                r = 2 * i
                return tuple(
                    jnp.maximum(regs[f],
                                jnp.maximum(rows_v[buf, r, pl.ds(f * 16, 16)],
                                            rows_v[buf, r + 1, pl.ds(f * 16, 16)]))
                    for f in range(NV))

            init = tuple(jnp.maximum(rows_v[buf, 0, pl.ds(f * 16, 16)],
                                     rows_v[buf, 1, pl.ds(f * 16, 16)])
                         for f in range(NV))
            regs = plsc.parallel_loop(1, C // 2, unroll=4, carry=init)(rbody)
            for f in range(NV):
                sl = pl.ds(f * 16, 16)
                acc_v[blo, sl] = jnp.maximum(acc_v[blo, sl], regs[f])

        @pl.when(nb > 0)
        def _slow():
            def run_fn(j, _):
                b = blo + j
                s_lo = seg_v[pl.ds(b, 16)][0]
                s_hi = seg_v[pl.ds(b + 1, 16)][0]
                lo = jnp.maximum(s_lo, gbase) - gbase
                hi = jnp.minimum(s_hi, gbase + C) - gbase

                @pl.when(hi > lo)
                def _run():
                    reduce_rows(buf, lo, hi, b)

                return 0

            lax.fori_loop(0, nb + 1, run_fn, 0)

    def pair_body(i, _):
        c = 2 * i
        copy(c, 0).wait()
        process(c, 0)

        @pl.when(c + 2 < NCHUNK)
        def _prefetch0():
            copy(c + 2, 0).start()

        copy(c + 1, 1).wait()
        process(c + 1, 1)

        @pl.when(c + 3 < NCHUNK)
        def _prefetch1():
            copy(c + 3, 1).start()

        return 0

    lax.fori_loop(0, NCHUNK // 2, pair_body, 0)

    pltpu.sync_copy(acc_v, out_hbm.at[wid])


@jax.jit
def _segmax_sc(point_feature, seg_starts):
    mesh = plsc.VectorSubcoreMesh(core_axis_name="c", subcore_axis_name="s")
    return pl.kernel(
        _segmax_sc_body,
        out_type=jax.ShapeDtypeStruct((NW, B, F), jnp.float32),
        mesh=mesh,
        scratch_types=[
            pltpu.VMEM((SEGPAD,), jnp.int32),
            pltpu.VMEM((2, C, F), jnp.float32),
            pltpu.VMEM((B, F), jnp.float32),
            pltpu.SemaphoreType.DMA,
            pltpu.SemaphoreType.DMA,
        ],
    )(point_feature, seg_starts)


def _mlp_body(part_ref, W1_ref, b1_ref, Ws1_ref, bs1_ref, g1_ref, beta1_ref,
              W2_ref, b2_ref, Ws2_ref, bs2_ref, g2_ref, beta2_ref,
              W3_ref, b3_ref, out_ref):
    maxx = jnp.max(part_ref[...], axis=0)

    def matT(x, w_ref):
        return lax.dot_general(x, w_ref[...], (((1,), (1,)), ((), ())),
                               preferred_element_type=jnp.float32)

    def bn(x, g, b):
        mu = jnp.mean(x, axis=0, keepdims=True)
        var = jnp.mean((x - mu) * (x - mu), axis=0, keepdims=True)
        return (x - mu) * lax.rsqrt(var + EPS) * g + b

    shortcut1 = matT(maxx, Ws1_ref) + bs1_ref[...]
    fc1 = jnp.maximum(matT(maxx, W1_ref) + b1_ref[...], 0.0)
    residual = bn(fc1, g1_ref[...], beta1_ref[...]) + shortcut1

    shortcut2 = matT(residual, Ws2_ref) + bs2_ref[...]
    x = bn(jnp.maximum(matT(residual, W2_ref) + b2_ref[...], 0.0),
           g2_ref[...], beta2_ref[...])
    x = x + shortcut2
    out_ref[...] = matT(x, W3_ref) + b3_ref[...]


@jax.jit
def _mlp_tc(partials, W1, b1, Ws1, bs1, g1, beta1,
            W2, b2, Ws2, bs2, g2, beta2, W3, b3):
    args = (partials, W1, b1[None, :], Ws1, bs1[None, :], g1[None, :],
            beta1[None, :], W2, b2[None, :], Ws2, bs2[None, :], g2[None, :],
            beta2[None, :], W3, b3[None, :])
    return pl.pallas_call(
        _mlp_body,
        out_shape=jax.ShapeDtypeStruct((B, 3), jnp.float32),
    )(*args)


def kernel(point_feature, voxel_feature, coords,
           W1, b1, Ws1, bs1, g1, beta1,
           W2, b2, Ws2, bs2, g2, beta2,
           W3, b3):
    ids = coords[:, 0].astype(jnp.int32)
    starts = jnp.searchsorted(ids, jnp.arange(B, dtype=jnp.int32))
    seg_starts = jnp.concatenate(
        [starts.astype(jnp.int32),
         jnp.full((SEGPAD - B,), N, dtype=jnp.int32)])
    partials = _segmax_sc(point_feature, seg_starts)
    return _mlp_tc(partials, W1, b1, Ws1, bs1, g1, beta1,
                   W2, b2, Ws2, bs2, g2, beta2, W3, b3)


# speedup vs baseline: 1.0011x; 1.0011x over previous
import functools

import jax
import jax.numpy as jnp
from jax import lax
from jax.experimental import pallas as pl
from jax.experimental.pallas import tpu as pltpu
from jax.experimental.pallas import tpu_sc as plsc

N = 32768
F = 256
B = 16
EPS = 1e-5

NW = 32
RPW = N // NW
C = 128
NCHUNK = RPW // C
NV = F // 16
SEGPAD = 48


def _segmax_sc_body(pf_hbm, ids_hbm, seg_hbm, out_hbm,
                    ids_v, seg_v, rows_v, acc_v, sem0, sem1):
    wid = lax.axis_index("s") * 2 + lax.axis_index("c")
    base = wid * RPW

    sems = (sem0, sem1)

    def copy(c, buf):
        return pltpu.make_async_copy(
            pf_hbm.at[pl.ds(base + c * C, C)], rows_v.at[buf], sems[buf])

    copy(0, 0).start()
    copy(1, 1).start()

    pltpu.sync_copy(ids_hbm.at[pl.ds(base, RPW)], ids_v)
    pltpu.sync_copy(seg_hbm, seg_v)

    neg = jnp.full((16,), -jnp.inf, dtype=jnp.float32)

    def init_row(b, _):
        for f in range(NV):
            acc_v[b, pl.ds(f * 16, 16)] = neg
        return 0

    lax.fori_loop(0, B, init_row, 0)

    def process(c, buf):
        r0 = c * C
        gbase = base + r0
        s0 = ids_v[pl.ds(r0, 16)][0]
        s1 = ids_v[pl.ds(r0 + C - 16, 16)][15]

        @pl.when(s0 == s1)
        def _fast():
            def rbody(i, regs):
                r = 2 * i
                return tuple(
                    jnp.maximum(regs[f],
                                jnp.maximum(rows_v[buf, r, pl.ds(f * 16, 16)],
                                            rows_v[buf, r + 1, pl.ds(f * 16, 16)]))
                    for f in range(NV))

            init = tuple(jnp.maximum(rows_v[buf, 0, pl.ds(f * 16, 16)],
                                     rows_v[buf, 1, pl.ds(f * 16, 16)])
                         for f in range(NV))
            regs = plsc.parallel_loop(1, C // 2, unroll=4, carry=init)(rbody)
            for f in range(NV):
                sl = pl.ds(f * 16, 16)
                acc_v[s0, sl] = jnp.maximum(acc_v[s0, sl], regs[f])

        @pl.when(s0 != s1)
        def _slow():
            def seg_fn(b, _):
                s_lo = seg_v[pl.ds(b, 16)][0]
                s_hi = seg_v[pl.ds(b + 1, 16)][0]
                lo = jnp.maximum(s_lo, gbase) - gbase
                hi = jnp.minimum(s_hi, gbase + C) - gbase

                @pl.when(hi > lo)
                def _run():
                    def rbody(r, regs):
                        return tuple(
                            jnp.maximum(regs[f],
                                        rows_v[buf, r, pl.ds(f * 16, 16)])
                            for f in range(NV))

                    init = tuple(rows_v[buf, lo, pl.ds(f * 16, 16)]
                                 for f in range(NV))
                    regs = lax.fori_loop(lo + 1, hi, rbody, init)
                    for f in range(NV):
                        sl = pl.ds(f * 16, 16)
                        acc_v[b, sl] = jnp.maximum(acc_v[b, sl], regs[f])

                return 0

            lax.fori_loop(s0, s1 + 1, seg_fn, 0)

    def pair_body(i, _):
        c = 2 * i
        copy(c, 0).wait()
        process(c, 0)

        @pl.when(c + 2 < NCHUNK)
        def _prefetch0():
            copy(c + 2, 0).start()

        copy(c + 1, 1).wait()
        process(c + 1, 1)

        @pl.when(c + 3 < NCHUNK)
        def _prefetch1():
            copy(c + 3, 1).start()

        return 0

    lax.fori_loop(0, NCHUNK // 2, pair_body, 0)

    pltpu.sync_copy(acc_v, out_hbm.at[wid])


@jax.jit
def _segmax_sc(point_feature, ids, seg_starts):
    mesh = plsc.VectorSubcoreMesh(core_axis_name="c", subcore_axis_name="s")
    return pl.kernel(
        _segmax_sc_body,
        out_type=jax.ShapeDtypeStruct((NW, B, F), jnp.float32),
        mesh=mesh,
        scratch_types=[
            pltpu.VMEM((RPW,), jnp.int32),
            pltpu.VMEM((SEGPAD,), jnp.int32),
            pltpu.VMEM((2, C, F), jnp.float32),
            pltpu.VMEM((B, F), jnp.float32),
            pltpu.SemaphoreType.DMA,
            pltpu.SemaphoreType.DMA,
        ],
    )(point_feature, ids, seg_starts)


def _mlp_body(part_ref, W1_ref, b1_ref, Ws1_ref, bs1_ref, g1_ref, beta1_ref,
              W2_ref, b2_ref, Ws2_ref, bs2_ref, g2_ref, beta2_ref,
              W3_ref, b3_ref, out_ref):
    maxx = jnp.max(part_ref[...], axis=0)

    def matT(x, w_ref):
        return lax.dot_general(x, w_ref[...], (((1,), (1,)), ((), ())),
                               preferred_element_type=jnp.float32)

    def bn(x, g, b):
        mu = jnp.mean(x, axis=0, keepdims=True)
        var = jnp.mean((x - mu) * (x - mu), axis=0, keepdims=True)
        return (x - mu) * lax.rsqrt(var + EPS) * g + b

    shortcut1 = matT(maxx, Ws1_ref) + bs1_ref[...]
    fc1 = jnp.maximum(matT(maxx, W1_ref) + b1_ref[...], 0.0)
    residual = bn(fc1, g1_ref[...], beta1_ref[...]) + shortcut1

    shortcut2 = matT(residual, Ws2_ref) + bs2_ref[...]
    x = bn(jnp.maximum(matT(residual, W2_ref) + b2_ref[...], 0.0),
           g2_ref[...], beta2_ref[...])
    x = x + shortcut2
    out_ref[...] = matT(x, W3_ref) + b3_ref[...]


@jax.jit
def _mlp_tc(partials, W1, b1, Ws1, bs1, g1, beta1,
            W2, b2, Ws2, bs2, g2, beta2, W3, b3):
    args = (partials, W1, b1[None, :], Ws1, bs1[None, :], g1[None, :],
            beta1[None, :], W2, b2[None, :], Ws2, bs2[None, :], g2[None, :],
            beta2[None, :], W3, b3[None, :])
    return pl.pallas_call(
        _mlp_body,
        out_shape=jax.ShapeDtypeStruct((B, 3), jnp.float32),
    )(*args)


def kernel(point_feature, voxel_feature, coords,
           W1, b1, Ws1, bs1, g1, beta1,
           W2, b2, Ws2, bs2, g2, beta2,
           W3, b3):
    ids = coords[:, 0].astype(jnp.int32)
    starts = jnp.searchsorted(ids, jnp.arange(B, dtype=jnp.int32))
    seg_starts = jnp.concatenate(
        [starts.astype(jnp.int32),
         jnp.full((SEGPAD - B,), N, dtype=jnp.int32)])
    partials = _segmax_sc(point_feature, ids, seg_starts)
    return _mlp_tc(partials, W1, b1, Ws1, bs1, g1, beta1,
                   W2, b2, Ws2, bs2, g2, beta2, W3, b3)


# speedup vs baseline: 1.4166x; 1.4152x over previous
import functools

import jax
import jax.numpy as jnp
from jax import lax
from jax.experimental import pallas as pl
from jax.experimental.pallas import tpu as pltpu
from jax.experimental.pallas import tpu_sc as plsc

N = 32768
F = 256
B = 16
EPS = 1e-5

NW = 32
RPW = N // NW
C = 128
NCHUNK = RPW // C
NV = F // 16
SEGPAD = 48


def _segmax_sc_body(pf_hbm, ids_hbm, seg_hbm, out_hbm,
                    ids_v, seg_v, rows_v, acc_v, sem0, sem1):
    wid = lax.axis_index("s") * 2 + lax.axis_index("c")
    base = wid * RPW

    sems = (sem0, sem1)

    def copy(c, buf):
        return pltpu.make_async_copy(
            pf_hbm.at[pl.ds(base + c * C, C)], rows_v.at[buf], sems[buf])

    copy(0, 0).start()
    copy(1, 1).start()

    pltpu.sync_copy(ids_hbm.at[pl.ds(base, RPW)], ids_v)
    pltpu.sync_copy(seg_hbm, seg_v)

    neg = jnp.full((16,), -jnp.inf, dtype=jnp.float32)

    def init_row(b, _):
        for f in range(NV):
            acc_v[b, pl.ds(f * 16, 16)] = neg
        return 0

    lax.fori_loop(0, B, init_row, 0)

    def process(c, buf):
        r0 = c * C
        gbase = base + r0
        s0 = ids_v[pl.ds(r0, 16)][0]
        s1 = ids_v[pl.ds(r0 + C - 16, 16)][15]

        @pl.when(s0 == s1)
        def _fast():
            def rbody(i, regs):
                r = 2 * i
                return tuple(
                    jnp.maximum(regs[f],
                                jnp.maximum(rows_v[buf, r, pl.ds(f * 16, 16)],
                                            rows_v[buf, r + 1, pl.ds(f * 16, 16)]))
                    for f in range(NV))

            init = tuple(jnp.maximum(rows_v[buf, 0, pl.ds(f * 16, 16)],
                                     rows_v[buf, 1, pl.ds(f * 16, 16)])
                         for f in range(NV))
            regs = plsc.parallel_loop(1, C // 2, unroll=4, carry=init)(rbody)
            for f in range(NV):
                sl = pl.ds(f * 16, 16)
                acc_v[s0, sl] = jnp.maximum(acc_v[s0, sl], regs[f])

        @pl.when(s0 != s1)
        def _slow():
            def seg_fn(b, _):
                s_lo = seg_v[pl.ds(b, 16)][0]
                s_hi = seg_v[pl.ds(b + 1, 16)][0]
                lo = jnp.maximum(s_lo, gbase) - gbase
                hi = jnp.minimum(s_hi, gbase + C) - gbase

                @pl.when(hi > lo)
                def _run():
                    def rbody(r, regs):
                        return tuple(
                            jnp.maximum(regs[f],
                                        rows_v[buf, r, pl.ds(f * 16, 16)])
                            for f in range(NV))

                    init = tuple(rows_v[buf, lo, pl.ds(f * 16, 16)]
                                 for f in range(NV))
                    regs = lax.fori_loop(lo + 1, hi, rbody, init)
                    for f in range(NV):
                        sl = pl.ds(f * 16, 16)
                        acc_v[b, sl] = jnp.maximum(acc_v[b, sl], regs[f])

                return 0

            lax.fori_loop(s0, s1 + 1, seg_fn, 0)

    def pair_body(i, _):
        c = 2 * i
        copy(c, 0).wait()
        process(c, 0)

        @pl.when(c + 2 < NCHUNK)
        def _prefetch0():
            copy(c + 2, 0).start()

        copy(c + 1, 1).wait()
        process(c + 1, 1)

        @pl.when(c + 3 < NCHUNK)
        def _prefetch1():
            copy(c + 3, 1).start()

        return 0

    lax.fori_loop(0, NCHUNK // 2, pair_body, 0)

    pltpu.sync_copy(acc_v, out_hbm.at[wid])


@jax.jit
def _segmax_sc(point_feature, ids, seg_starts):
    mesh = plsc.VectorSubcoreMesh(core_axis_name="c", subcore_axis_name="s")
    return pl.kernel(
        _segmax_sc_body,
        out_type=jax.ShapeDtypeStruct((NW, B, F), jnp.float32),
        mesh=mesh,
        scratch_types=[
            pltpu.VMEM((RPW,), jnp.int32),
            pltpu.VMEM((SEGPAD,), jnp.int32),
            pltpu.VMEM((2, C, F), jnp.float32),
            pltpu.VMEM((B, F), jnp.float32),
            pltpu.SemaphoreType.DMA,
            pltpu.SemaphoreType.DMA,
        ],
    )(point_feature, ids, seg_starts)


def _mlp_body(part_ref, W1_ref, b1_ref, Ws1_ref, bs1_ref, g1_ref, beta1_ref,
              W2_ref, b2_ref, Ws2_ref, bs2_ref, g2_ref, beta2_ref,
              W3_ref, b3_ref, out_ref):
    maxx = jnp.max(part_ref[...], axis=0)

    def matT(x, w_ref):
        return lax.dot_general(x, w_ref[...], (((1,), (1,)), ((), ())),
                               preferred_element_type=jnp.float32)

    def bn(x, g, b):
        mu = jnp.mean(x, axis=0, keepdims=True)
        var = jnp.mean((x - mu) * (x - mu), axis=0, keepdims=True)
        return (x - mu) * lax.rsqrt(var + EPS) * g + b

    shortcut1 = matT(maxx, Ws1_ref) + bs1_ref[...]
    fc1 = jnp.maximum(matT(maxx, W1_ref) + b1_ref[...], 0.0)
    residual = bn(fc1, g1_ref[...], beta1_ref[...]) + shortcut1

    shortcut2 = matT(residual, Ws2_ref) + bs2_ref[...]
    x = bn(jnp.maximum(matT(residual, W2_ref) + b2_ref[...], 0.0),
           g2_ref[...], beta2_ref[...])
    x = x + shortcut2
    out_ref[...] = matT(x, W3_ref) + b3_ref[...]


@jax.jit
def _mlp_tc(partials, W1, b1, Ws1, bs1, g1, beta1,
            W2, b2, Ws2, bs2, g2, beta2, W3, b3):
    args = (partials, W1, b1[None, :], Ws1, bs1[None, :], g1[None, :],
            beta1[None, :], W2, b2[None, :], Ws2, bs2[None, :], g2[None, :],
            beta2[None, :], W3, b3[None, :])
    return pl.pallas_call(
        _mlp_body,
        out_shape=jax.ShapeDtypeStruct((B, 3), jnp.float32),
    )(*args)


def kernel(point_feature, voxel_feature, coords,
           W1, b1, Ws1, bs1, g1, beta1,
           W2, b2, Ws2, bs2, g2, beta2,
           W3, b3):
    ids = coords[:, 0].astype(jnp.int32)
    counts = jnp.sum(
        (ids[:, None] == jnp.arange(B, dtype=jnp.int32)[None, :])
        .astype(jnp.int32), axis=0)
    seg_starts = jnp.concatenate(
        [jnp.zeros((1,), jnp.int32),
         jnp.cumsum(counts).astype(jnp.int32),
         jnp.full((SEGPAD - B - 1,), N, dtype=jnp.int32)])
    partials = _segmax_sc(point_feature, ids, seg_starts)
    return _mlp_tc(partials, W1, b1, Ws1, bs1, g1, beta1,
                   W2, b2, Ws2, bs2, g2, beta2, W3, b3)


# speedup vs baseline: 1.4211x; 1.0032x over previous
import functools

import jax
import jax.numpy as jnp
from jax import lax
from jax.experimental import pallas as pl
from jax.experimental.pallas import tpu as pltpu
from jax.experimental.pallas import tpu_sc as plsc

N = 32768
F = 256
B = 16
EPS = 1e-5

NW = 32
RPW = N // NW
C = 128
NCHUNK = RPW // C
NV = F // 16
SEGPAD = 48


def _segmax_sc_body(pf_hbm, ids_hbm, seg_hbm, out_hbm,
                    ids_v, seg_v, rows_v, acc_v, sem0, sem1):
    wid = lax.axis_index("s") * 2 + lax.axis_index("c")
    base = wid * RPW

    sems = (sem0, sem1)

    def copy(c, buf):
        return pltpu.make_async_copy(
            pf_hbm.at[pl.ds(base + c * C, C)], rows_v.at[buf], sems[buf])

    copy(0, 0).start()
    copy(1, 1).start()

    pltpu.sync_copy(ids_hbm.at[pl.ds(base, RPW)], ids_v)
    pltpu.sync_copy(seg_hbm, seg_v)

    neg = jnp.full((16,), -jnp.inf, dtype=jnp.float32)

    def init_row(b, _):
        for f in range(NV):
            acc_v[b, pl.ds(f * 16, 16)] = neg
        return 0

    lax.fori_loop(0, B, init_row, 0)

    def process(c, buf):
        r0 = c * C
        gbase = base + r0
        s0 = ids_v[pl.ds(r0, 16)][0]
        s1 = ids_v[pl.ds(r0 + C - 16, 16)][15]

        @pl.when(s0 == s1)
        def _fast():
            def rbody(i, regs):
                r = 2 * i
                return tuple(
                    jnp.maximum(regs[f],
                                jnp.maximum(rows_v[buf, r, pl.ds(f * 16, 16)],
                                            rows_v[buf, r + 1, pl.ds(f * 16, 16)]))
                    for f in range(NV))

            init = tuple(jnp.maximum(rows_v[buf, 0, pl.ds(f * 16, 16)],
                                     rows_v[buf, 1, pl.ds(f * 16, 16)])
                         for f in range(NV))
            regs = plsc.parallel_loop(1, C // 2, unroll=8, carry=init)(rbody)
            for f in range(NV):
                sl = pl.ds(f * 16, 16)
                acc_v[s0, sl] = jnp.maximum(acc_v[s0, sl], regs[f])

        @pl.when(s0 != s1)
        def _slow():
            def seg_fn(b, _):
                s_lo = seg_v[pl.ds(b, 16)][0]
                s_hi = seg_v[pl.ds(b + 1, 16)][0]
                lo = jnp.maximum(s_lo, gbase) - gbase
                hi = jnp.minimum(s_hi, gbase + C) - gbase

                @pl.when(hi > lo)
                def _run():
                    def rbody(r, regs):
                        return tuple(
                            jnp.maximum(regs[f],
                                        rows_v[buf, r, pl.ds(f * 16, 16)])
                            for f in range(NV))

                    init = tuple(rows_v[buf, lo, pl.ds(f * 16, 16)]
                                 for f in range(NV))
                    regs = lax.fori_loop(lo + 1, hi, rbody, init)
                    for f in range(NV):
                        sl = pl.ds(f * 16, 16)
                        acc_v[b, sl] = jnp.maximum(acc_v[b, sl], regs[f])

                return 0

            lax.fori_loop(s0, s1 + 1, seg_fn, 0)

    def pair_body(i, _):
        c = 2 * i
        copy(c, 0).wait()
        process(c, 0)

        @pl.when(c + 2 < NCHUNK)
        def _prefetch0():
            copy(c + 2, 0).start()

        copy(c + 1, 1).wait()
        process(c + 1, 1)

        @pl.when(c + 3 < NCHUNK)
        def _prefetch1():
            copy(c + 3, 1).start()

        return 0

    lax.fori_loop(0, NCHUNK // 2, pair_body, 0)

    pltpu.sync_copy(acc_v, out_hbm.at[wid])


@jax.jit
def _segmax_sc(point_feature, ids, seg_starts):
    mesh = plsc.VectorSubcoreMesh(core_axis_name="c", subcore_axis_name="s")
    return pl.kernel(
        _segmax_sc_body,
        out_type=jax.ShapeDtypeStruct((NW, B, F), jnp.float32),
        mesh=mesh,
        scratch_types=[
            pltpu.VMEM((RPW,), jnp.int32),
            pltpu.VMEM((SEGPAD,), jnp.int32),
            pltpu.VMEM((2, C, F), jnp.float32),
            pltpu.VMEM((B, F), jnp.float32),
            pltpu.SemaphoreType.DMA,
            pltpu.SemaphoreType.DMA,
        ],
    )(point_feature, ids, seg_starts)


def _mlp_body(part_ref, W1_ref, b1_ref, Ws1_ref, bs1_ref, g1_ref, beta1_ref,
              W2_ref, b2_ref, Ws2_ref, bs2_ref, g2_ref, beta2_ref,
              W3_ref, b3_ref, out_ref):
    maxx = jnp.max(part_ref[...], axis=0)

    def matT(x, w_ref):
        return lax.dot_general(x, w_ref[...], (((1,), (1,)), ((), ())),
                               preferred_element_type=jnp.float32)

    def bn(x, g, b):
        mu = jnp.mean(x, axis=0, keepdims=True)
        var = jnp.mean((x - mu) * (x - mu), axis=0, keepdims=True)
        return (x - mu) * lax.rsqrt(var + EPS) * g + b

    shortcut1 = matT(maxx, Ws1_ref) + bs1_ref[...]
    fc1 = jnp.maximum(matT(maxx, W1_ref) + b1_ref[...], 0.0)
    residual = bn(fc1, g1_ref[...], beta1_ref[...]) + shortcut1

    shortcut2 = matT(residual, Ws2_ref) + bs2_ref[...]
    x = bn(jnp.maximum(matT(residual, W2_ref) + b2_ref[...], 0.0),
           g2_ref[...], beta2_ref[...])
    x = x + shortcut2
    out_ref[...] = matT(x, W3_ref) + b3_ref[...]


@jax.jit
def _mlp_tc(partials, W1, b1, Ws1, bs1, g1, beta1,
            W2, b2, Ws2, bs2, g2, beta2, W3, b3):
    args = (partials, W1, b1[None, :], Ws1, bs1[None, :], g1[None, :],
            beta1[None, :], W2, b2[None, :], Ws2, bs2[None, :], g2[None, :],
            beta2[None, :], W3, b3[None, :])
    return pl.pallas_call(
        _mlp_body,
        out_shape=jax.ShapeDtypeStruct((B, 3), jnp.float32),
    )(*args)


def kernel(point_feature, voxel_feature, coords,
           W1, b1, Ws1, bs1, g1, beta1,
           W2, b2, Ws2, bs2, g2, beta2,
           W3, b3):
    ids = coords[:, 0].astype(jnp.int32)
    counts = jnp.sum(
        (ids[:, None] == jnp.arange(B, dtype=jnp.int32)[None, :])
        .astype(jnp.int32), axis=0)
    seg_starts = jnp.concatenate(
        [jnp.zeros((1,), jnp.int32),
         jnp.cumsum(counts).astype(jnp.int32),
         jnp.full((SEGPAD - B - 1,), N, dtype=jnp.int32)])
    partials = _segmax_sc(point_feature, ids, seg_starts)
    return _mlp_tc(partials, W1, b1, Ws1, bs1, g1, beta1,
                   W2, b2, Ws2, bs2, g2, beta2, W3, b3)
